# Initial kernel scaffold; baseline (speedup 1.0000x reference)
import jax
import jax.numpy as jnp
from jax.experimental import pallas as pl


def kernel(z):
    raise NotImplementedError("write your pallas kernel here")


# speedup vs baseline: 11.8707x; 11.8707x over previous
import functools

import jax
import jax.numpy as jnp
from jax import lax
from jax.experimental import pallas as pl
from jax.experimental.pallas import tpu as pltpu
from jax.experimental.pallas import tpu_sc as plsc

N_ROWS = 128
N_COLS = 32768
L = 16
N_WORKERS = 32
ROWS_PER_W = N_ROWS // N_WORKERS
NVEC = N_COLS // L


def _row_sparsemax(row_v, cand_v):
    def max_body(i, acc):
        return jnp.maximum(acc, row_v[pl.ds(i * L, L)])

    mx_vec = lax.fori_loop(
        0, NVEC, max_body, jnp.full((L,), -jnp.inf, jnp.float32))
    m = jnp.max(mx_vec)
    thr = m - 1.0

    def comp_body(i, off):
        v = row_v[pl.ds(i * L, L)]
        msk = v > thr
        plsc.store_compressed(cand_v.at[pl.ds(off, L)], v, mask=msk)
        return off + jnp.sum(jnp.where(msk, 1, 0))

    c0 = lax.fori_loop(0, NVEC, comp_body, jnp.int32(0))
    cand_v[pl.ds(c0, L)] = jnp.full((L,), thr - 1.0, jnp.float32)
    ncv = lax.shift_right_logical(c0 + (L - 1), 4)

    def f_eval(tau):
        def nb(i, carry):
            s_acc, c_acc = carry
            a = cand_v[pl.ds(i * L, L)] - m
            msk = a > tau
            return (s_acc + jnp.where(msk, a, 0.0),
                    c_acc + jnp.where(msk, 1.0, 0.0))

        s_vec, c_vec = lax.fori_loop(
            0, ncv, nb,
            (jnp.zeros((L,), jnp.float32), jnp.zeros((L,), jnp.float32)))
        return jnp.sum(s_vec), jnp.sum(c_vec)

    def cond(st):
        tau_prev, tau_cur, it = st
        return (tau_cur > tau_prev) & (it < 64)

    def body(st):
        _, tau_cur, it = st
        s, c = f_eval(tau_cur)
        tau_next = (jnp.full((L,), s - 1.0) / jnp.full((L,), c))[0]
        return tau_cur, tau_next, it + 1

    tau_prev, tau_cur, _ = lax.while_loop(
        cond, body, (jnp.float32(-2.0), jnp.float32(-1.0), jnp.int32(0)))
    tau = jnp.maximum(tau_prev, tau_cur)

    th2 = m + tau

    def proj_body(i, carry):
        v = row_v[pl.ds(i * L, L)]
        row_v[pl.ds(i * L, L)] = jnp.maximum(v - th2, 0.0)
        return carry

    lax.fori_loop(0, NVEC, proj_body, jnp.int32(0))


def kernel(z):
    mesh = plsc.VectorSubcoreMesh(core_axis_name="c", subcore_axis_name="s")

    @functools.partial(
        pl.kernel,
        out_type=jax.ShapeDtypeStruct((N_ROWS, N_COLS), jnp.float32),
        mesh=mesh,
        scratch_types=[
            pltpu.VMEM((N_COLS,), jnp.float32),
            pltpu.VMEM((N_COLS + L,), jnp.float32),
        ],
        compiler_params=pltpu.CompilerParams(needs_layout_passes=False),
    )
    def sc_kernel(z_hbm, out_hbm, row_v, cand_v):
        wid = lax.axis_index("s") * 2 + lax.axis_index("c")
        base = wid * ROWS_PER_W

        def row_body(r, carry):
            row = base + r
            pltpu.sync_copy(z_hbm.at[row], row_v)
            _row_sparsemax(row_v, cand_v)
            pltpu.sync_copy(row_v, out_hbm.at[row])
            return carry

        lax.fori_loop(0, ROWS_PER_W, row_body, jnp.int32(0))

    return sc_kernel(z)


# speedup vs baseline: 16.7208x; 1.4086x over previous
import functools

import jax
import jax.numpy as jnp
from jax import lax
from jax.experimental import pallas as pl
from jax.experimental.pallas import tpu as pltpu
from jax.experimental.pallas import tpu_sc as plsc

N_ROWS = 128
N_COLS = 32768
L = 16
N_WORKERS = 32
ROWS_PER_W = N_ROWS // N_WORKERS
NVEC = N_COLS // L
U = 8


def _row_sparsemax(row_v, zero_v, cand_idx):
    lanes = lax.iota(jnp.int32, L)

    def fuse_body(i, carry):
        acc, off = carry
        for u in range(U):
            j = i * U + u
            v = row_v[pl.ds(j * L, L)]
            acc = jnp.maximum(acc, v)
            msk = v > acc - 1.0
            plsc.store_compressed(
                cand_idx.at[pl.ds(off, L)], lanes + j * L, mask=msk)
            off = off + plsc.all_reduce_population_count(msk)[0]
        return acc, off

    acc, c0 = lax.fori_loop(
        0, NVEC // U, fuse_body,
        (jnp.full((L,), -jnp.inf, jnp.float32), jnp.int32(0)))
    m = jnp.max(acc)
    cand_idx[pl.ds(c0, L)] = jnp.full((L,), N_COLS, jnp.int32)
    ncv = lax.shift_right_logical(c0 + (L - 1), 4)

    def f_eval(tau):
        def nb(i, carry):
            s_acc, c_acc = carry
            a = plsc.load_gather(row_v, [cand_idx[pl.ds(i * L, L)]]) - m
            msk = a > tau
            return (s_acc + jnp.where(msk, a, 0.0),
                    c_acc + jnp.where(msk, 1.0, 0.0))

        s_vec, c_vec = lax.fori_loop(
            0, ncv, nb,
            (jnp.zeros((L,), jnp.float32), jnp.zeros((L,), jnp.float32)))
        return jnp.sum(s_vec), jnp.sum(c_vec)

    def cond(st):
        tau_prev, tau_cur, it = st
        return (tau_cur > tau_prev) & (it < 64)

    def body(st):
        _, tau_cur, it = st
        s, c = f_eval(tau_cur)
        tau_next = (jnp.full((L,), s - 1.0) / jnp.full((L,), c))[0]
        return tau_cur, tau_next, it + 1

    tau_prev, tau_cur, _ = lax.while_loop(
        cond, body, (jnp.float32(-2.0), jnp.float32(-1.0), jnp.int32(0)))
    tau = jnp.maximum(tau_prev, tau_cur)

    th2 = m + tau

    def sc_body(i, carry):
        idxv = cand_idx[pl.ds(i * L, L)]
        p = jnp.maximum(plsc.load_gather(row_v, [idxv]) - th2, 0.0)
        plsc.store_scatter(zero_v, [idxv], p)
        return carry

    lax.fori_loop(0, ncv, sc_body, jnp.int32(0))
    return ncv


def _rezero(zero_v, cand_idx, ncv):
    zvec = jnp.zeros((L,), jnp.float32)

    def rz_body(i, carry):
        plsc.store_scatter(zero_v, [cand_idx[pl.ds(i * L, L)]], zvec)
        return carry

    lax.fori_loop(0, ncv, rz_body, jnp.int32(0))


def kernel(z):
    mesh = plsc.VectorSubcoreMesh(core_axis_name="c", subcore_axis_name="s")

    @functools.partial(
        pl.kernel,
        out_type=jax.ShapeDtypeStruct((N_ROWS, N_COLS), jnp.float32),
        mesh=mesh,
        scratch_types=[
            pltpu.VMEM((N_COLS + L,), jnp.float32),
            pltpu.VMEM((N_COLS + L,), jnp.float32),
            pltpu.VMEM((N_COLS + L,), jnp.int32),
        ],
        compiler_params=pltpu.CompilerParams(needs_layout_passes=False),
    )
    def sc_kernel(z_hbm, out_hbm, row_v, zero_v, cand_idx):
        wid = lax.axis_index("s") * 2 + lax.axis_index("c")
        base = wid * ROWS_PER_W

        zvec = jnp.zeros((L,), jnp.float32)

        def zb(i, carry):
            for u in range(U):
                zero_v[pl.ds((i * U + u) * L, L)] = zvec
            return carry

        lax.fori_loop(0, NVEC // U, zb, jnp.int32(0))
        zero_v[pl.ds(N_COLS, L)] = zvec
        row_v[pl.ds(N_COLS, L)] = jnp.full((L,), -jnp.inf, jnp.float32)

        def row_body(r, carry):
            row = base + r
            pltpu.sync_copy(z_hbm.at[row], row_v.at[pl.ds(0, N_COLS)])
            ncv = _row_sparsemax(row_v, zero_v, cand_idx)
            pltpu.sync_copy(zero_v.at[pl.ds(0, N_COLS)], out_hbm.at[row])
            _rezero(zero_v, cand_idx, ncv)
            return carry

        lax.fori_loop(0, ROWS_PER_W, row_body, jnp.int32(0))

    return sc_kernel(z)
